# Initial kernel scaffold; baseline (speedup 1.0000x reference)
import jax
import jax.numpy as jnp
from jax.experimental import pallas as pl


def kernel(point_coordinates1, point_coordinates2, features1, features2, W1, b1, g1, be1, W2, b2, g2, be2):
    raise NotImplementedError("write your pallas kernel here")


# speedup vs baseline: 16.6136x; 16.6136x over previous
import functools

import jax
import jax.numpy as jnp
from jax.experimental import pallas as pl
from jax.experimental.pallas import tpu as pltpu

EPS_BN = 1e-5
NT = 512


def _k1_body(xyz1_ref, xyz2_ref, f1_ref, f2_ref, w1a_ref, w1b_ref, b1_ref,
             y1_ref, stats_ref):
    S = xyz2_ref.shape[-1]
    nt = xyz1_ref.shape[-1]
    x1 = xyz1_ref[0]
    x2 = xyz2_ref[0]
    m = jax.lax.dot_general(x1, x2, (((0,), (0,)), ((), ())),
                            preferred_element_type=jnp.float32)
    n1 = (x1[0] * x1[0] + x1[1] * x1[1]) + x1[2] * x1[2]
    n2 = (x2[0] * x2[0] + x2[1] * x2[1]) + x2[2] * x2[2]
    d = (-2.0 * m + n1[:, None]) + n2[None, :]

    iota = jax.lax.broadcasted_iota(jnp.int32, (nt, S), 1)
    idxs = []
    vals = []
    for _ in range(3):
        mn = jnp.min(d, axis=1)
        ii = jnp.min(jnp.where(d == mn[:, None], iota, S), axis=1)
        idxs.append(ii)
        vals.append(mn)
        d = jnp.where(iota == ii[:, None], jnp.float32(jnp.inf), d)

    r0 = 1.0 / (vals[0] + 1e-8)
    r1 = 1.0 / (vals[1] + 1e-8)
    r2 = 1.0 / (vals[2] + 1e-8)
    rs = r0 + r1 + r2
    w0 = r0 / rs
    w1 = r1 / rs
    w2 = r2 / rs

    iota_s = jax.lax.broadcasted_iota(jnp.int32, (S, nt), 0)
    at = jnp.where(iota_s == idxs[0][None, :], w0[None, :], 0.0)
    at = at + jnp.where(iota_s == idxs[1][None, :], w1[None, :], 0.0)
    at = at + jnp.where(iota_s == idxs[2][None, :], w2[None, :], 0.0)

    f2 = f2_ref[0]
    interp = jax.lax.dot_general(f2, at, (((1,), (0,)), ((), ())),
                                 preferred_element_type=jnp.float32,
                                 precision=jax.lax.Precision.HIGHEST)

    f1 = f1_ref[0]
    y = jax.lax.dot_general(w1a_ref[...], f1, (((1,), (0,)), ((), ())),
                            preferred_element_type=jnp.float32)
    y = y + jax.lax.dot_general(w1b_ref[...], interp, (((1,), (0,)), ((), ())),
                                preferred_element_type=jnp.float32)
    y = y + b1_ref[...]
    y1_ref[0] = y

    b = pl.program_id(0)
    n = pl.program_id(1)

    @pl.when(jnp.logical_and(b == 0, n == 0))
    def _():
        stats_ref[...] = jnp.zeros_like(stats_ref)

    s = jnp.sum(y, axis=1)
    q = jnp.sum(y * y, axis=1)
    stats_ref[...] += jnp.stack([s, q], axis=0)


def _k2_body(y1_ref, scale_ref, shift_ref, w2_ref, b2_ref, y2_ref, stats_ref):
    x = y1_ref[0]
    x = jnp.maximum(x * scale_ref[...] + shift_ref[...], 0.0)
    y = jax.lax.dot_general(w2_ref[...], x, (((1,), (0,)), ((), ())),
                            preferred_element_type=jnp.float32)
    y = y + b2_ref[...]
    y2_ref[0] = y

    b = pl.program_id(0)
    n = pl.program_id(1)

    @pl.when(jnp.logical_and(b == 0, n == 0))
    def _():
        stats_ref[...] = jnp.zeros_like(stats_ref)

    s = jnp.sum(y, axis=1)
    q = jnp.sum(y * y, axis=1)
    stats_ref[...] += jnp.stack([s, q], axis=0)


def _k3_body(y2_ref, scale_ref, shift_ref, out_ref):
    out_ref[0] = jnp.maximum(y2_ref[0] * scale_ref[...] + shift_ref[...], 0.0)


def kernel(point_coordinates1, point_coordinates2, features1, features2,
           W1, b1, g1, be1, W2, b2, g2, be2):
    B, _, N = point_coordinates1.shape
    S = point_coordinates2.shape[-1]
    C1 = features1.shape[1]
    C2 = features2.shape[1]
    CM1 = W1.shape[0]
    CM2 = W2.shape[0]
    nb = N // NT
    Q = B * N

    w1a = W1[:, :C1]
    w1b = W1[:, C1:]
    b1c = b1[:, None]
    b2c = b2[:, None]

    y1, stats1 = pl.pallas_call(
        _k1_body,
        grid=(B, nb),
        in_specs=[
            pl.BlockSpec((1, 3, NT), lambda b, n: (b, 0, n)),
            pl.BlockSpec((1, 3, S), lambda b, n: (b, 0, 0)),
            pl.BlockSpec((1, C1, NT), lambda b, n: (b, 0, n)),
            pl.BlockSpec((1, C2, S), lambda b, n: (b, 0, 0)),
            pl.BlockSpec((CM1, C1), lambda b, n: (0, 0)),
            pl.BlockSpec((CM1, C2), lambda b, n: (0, 0)),
            pl.BlockSpec((CM1, 1), lambda b, n: (0, 0)),
        ],
        out_specs=[
            pl.BlockSpec((1, CM1, NT), lambda b, n: (b, 0, n)),
            pl.BlockSpec((2, CM1), lambda b, n: (0, 0)),
        ],
        out_shape=[
            jax.ShapeDtypeStruct((B, CM1, N), jnp.float32),
            jax.ShapeDtypeStruct((2, CM1), jnp.float32),
        ],
        compiler_params=pltpu.CompilerParams(
            dimension_semantics=("arbitrary", "arbitrary")),
    )(point_coordinates1, point_coordinates2, features1, features2,
      w1a, w1b, b1c)

    mean1 = stats1[0] / Q
    var1 = stats1[1] / Q - mean1 * mean1
    sc1 = g1 / jnp.sqrt(var1 + EPS_BN)
    sh1 = be1 - mean1 * sc1

    y2, stats2 = pl.pallas_call(
        _k2_body,
        grid=(B, nb),
        in_specs=[
            pl.BlockSpec((1, CM1, NT), lambda b, n: (b, 0, n)),
            pl.BlockSpec((CM1, 1), lambda b, n: (0, 0)),
            pl.BlockSpec((CM1, 1), lambda b, n: (0, 0)),
            pl.BlockSpec((CM2, CM1), lambda b, n: (0, 0)),
            pl.BlockSpec((CM2, 1), lambda b, n: (0, 0)),
        ],
        out_specs=[
            pl.BlockSpec((1, CM2, NT), lambda b, n: (b, 0, n)),
            pl.BlockSpec((2, CM2), lambda b, n: (0, 0)),
        ],
        out_shape=[
            jax.ShapeDtypeStruct((B, CM2, N), jnp.float32),
            jax.ShapeDtypeStruct((2, CM2), jnp.float32),
        ],
        compiler_params=pltpu.CompilerParams(
            dimension_semantics=("arbitrary", "arbitrary")),
    )(y1, sc1[:, None], sh1[:, None], W2, b2c)

    mean2 = stats2[0] / Q
    var2 = stats2[1] / Q - mean2 * mean2
    sc2 = g2 / jnp.sqrt(var2 + EPS_BN)
    sh2 = be2 - mean2 * sc2

    out = pl.pallas_call(
        _k3_body,
        grid=(B,),
        in_specs=[
            pl.BlockSpec((1, CM2, N), lambda b: (b, 0, 0)),
            pl.BlockSpec((CM2, 1), lambda b: (0, 0)),
            pl.BlockSpec((CM2, 1), lambda b: (0, 0)),
        ],
        out_specs=pl.BlockSpec((1, CM2, N), lambda b: (b, 0, 0)),
        out_shape=jax.ShapeDtypeStruct((B, CM2, N), jnp.float32),
    )(y2, sc2[:, None], sh2[:, None])

    return out
